# baseline (speedup 1.0000x reference)
import functools

import jax
import jax.numpy as jnp
from jax import lax
from jax.experimental import pallas as pl
from jax.experimental.pallas import tpu as pltpu
from jax.experimental.pallas import tpu_sc as plsc

N = 10000
E = 320000
D = 128

NC = 2
NS = 16
NW = NC * NS
L = 16

K = 125
NB = 80
EPT = NB * K
NCH = 4
CHB = NB // NCH
NT = CHB // 2

N_PAD = 10240
RPT = N_PAD // NS
SR = 64
HR = N_PAD // D

_MESH = plsc.VectorSubcoreMesh(
    core_axis_name="c", subcore_axis_name="s", num_cores=NC, num_subcores=NS)
_SC_PARAMS = pltpu.CompilerParams(needs_layout_passes=False)


@functools.partial(
    pl.kernel,
    out_type=jax.ShapeDtypeStruct((NC, HR, D), jnp.float32),
    mesh=_MESH,
    scratch_types=[
        pltpu.VMEM((HR, D), jnp.float32),
        pltpu.VMEM((EPT,), jnp.int32),
        pltpu.VMEM((HR,), jnp.int32),
        pltpu.VMEM_SHARED((HR, D), jnp.float32),
    ],
    compiler_params=_SC_PARAMS,
)
def _deg_kernel(dst_hbm, out_hbm, hist_v, idx_v, rows_i, hist_sh):
    cid = lax.axis_index("c")
    sid = lax.axis_index("s")
    wid = cid * NS + sid

    zeros16 = jnp.zeros((L,), jnp.float32)
    ones16 = jnp.ones((L,), jnp.float32)
    iota16 = lax.iota(jnp.int32, L)

    def zero_row(r, carry):
        def zero_col(col, c2):
            hist_v[r, pl.ds(col * L, L)] = zeros16
            return c2
        return lax.fori_loop(0, D // L, zero_col, carry)

    lax.fori_loop(0, HR, zero_row, 0)

    def iota_body(i, carry):
        rows_i[pl.ds(i * L, L)] = iota16 + i * L
        return carry

    lax.fori_loop(0, HR // L, iota_body, 0)

    @pl.when(sid == 0)
    def _():
        pltpu.sync_copy(hist_v, hist_sh)

    pltpu.sync_copy(dst_hbm.at[pl.ds(wid * EPT, EPT)], idx_v)

    def acc_body(j, carry):
        idx = idx_v[pl.ds(j * L, L)]
        plsc.addupdate_scatter(hist_v, [idx >> 7, idx & 127], ones16)
        return carry

    lax.fori_loop(0, EPT // L, acc_body, 0)

    plsc.subcore_barrier()
    pltpu.sync_copy(hist_v, hist_sh.at[rows_i], add=True)
    plsc.subcore_barrier()

    @pl.when(sid == 0)
    def _():
        pltpu.sync_copy(hist_sh, out_hbm.at[cid])


def _mm_body(x_ref, w_ref, degp_ref, g_ref, dis_ref):
    deg = jnp.sum(degp_ref[...], axis=0) + 1.0
    dis = lax.rsqrt(deg)
    h = jnp.dot(x_ref[...], w_ref[...], preferred_element_type=jnp.float32)
    g_ref[...] = h * dis
    dis_ref[...] = dis


def _matmul_scale(x, w, degp3):
    BM = 2000
    grid = (N // BM,)
    return pl.pallas_call(
        _mm_body,
        grid=grid,
        in_specs=[
            pl.BlockSpec((BM, D), lambda i: (i, 0)),
            pl.BlockSpec((D, D), lambda i: (0, 0)),
            pl.BlockSpec((NC, BM, 1), lambda i: (0, i, 0)),
        ],
        out_specs=[
            pl.BlockSpec((BM, D), lambda i: (i, 0)),
            pl.BlockSpec((BM, 1), lambda i: (i, 0)),
        ],
        out_shape=[
            jax.ShapeDtypeStruct((N_PAD, D), jnp.float32),
            jax.ShapeDtypeStruct((N, 1), jnp.float32),
        ],
    )(x, w, degp3)


@functools.partial(
    pl.kernel,
    out_type=jax.ShapeDtypeStruct((NC, N_PAD, D), jnp.float32),
    mesh=_MESH,
    scratch_types=[
        pltpu.VMEM_SHARED((N_PAD, D), jnp.float32),
        pltpu.VMEM((CHB, K), jnp.int32),
        pltpu.VMEM((CHB, K), jnp.int32),
        pltpu.VMEM((K, D), jnp.float32),
        pltpu.VMEM((K, D), jnp.float32),
        pltpu.SemaphoreType.DMA,
        pltpu.SemaphoreType.DMA,
    ],
    compiler_params=_SC_PARAMS,
)
def _agg_kernel(g_hbm, src_hbm, dst_hbm, out_hbm,
                acc_sh, src_t, dst_t, rows_a, rows_b, gsem_a, gsem_b):
    cid = lax.axis_index("c")
    sid = lax.axis_index("s")
    wid = cid * NS + sid
    row0 = sid * RPT

    @pl.when(cid == 0)
    def _():
        for j in range(RPT // SR):
            pltpu.sync_copy(g_hbm.at[pl.ds(row0 + j * SR, SR)],
                            rows_a.at[pl.ds(0, SR)])
            pltpu.sync_copy(rows_a.at[pl.ds(0, SR)],
                            acc_sh.at[pl.ds(row0 + j * SR, SR)])

    @pl.when(cid == 1)
    def _():
        zeros16 = jnp.zeros((L,), jnp.float32)

        def zrow(r, carry):
            def zcol(col, c2):
                rows_a[r, pl.ds(col * L, L)] = zeros16
                return c2
            return lax.fori_loop(0, D // L, zcol, carry)

        lax.fori_loop(0, SR, zrow, 0)
        for j in range(RPT // SR):
            pltpu.sync_copy(rows_a.at[pl.ds(0, SR)],
                            acc_sh.at[pl.ds(row0 + j * SR, SR)])

    plsc.subcore_barrier()

    for c in range(NCH):
        pltpu.sync_copy(src_hbm.at[wid, c], src_t)
        pltpu.sync_copy(dst_hbm.at[wid, c], dst_t)
        pltpu.async_copy(g_hbm.at[src_t.at[0]], rows_a, gsem_a)

        def edge_body(t, carry):
            ja = 2 * t
            jb = 2 * t + 1
            pltpu.make_async_copy(
                g_hbm.at[src_t.at[ja]], rows_a, gsem_a).wait()
            pltpu.async_copy(g_hbm.at[src_t.at[jb]], rows_b, gsem_b)
            pltpu.sync_copy(rows_a, acc_sh.at[dst_t.at[ja]], add=True)
            pltpu.make_async_copy(
                g_hbm.at[src_t.at[jb]], rows_b, gsem_b).wait()

            @pl.when(t + 1 < NT)
            def _():
                pltpu.async_copy(g_hbm.at[src_t.at[ja + 2]], rows_a, gsem_a)

            pltpu.sync_copy(rows_b, acc_sh.at[dst_t.at[jb]], add=True)
            return carry

        lax.fori_loop(0, NT, edge_body, 0)

    plsc.subcore_barrier()

    pltpu.sync_copy(acc_sh.at[pl.ds(row0, RPT)],
                    out_hbm.at[cid, pl.ds(row0, RPT)])


def _fin_body(aggp_ref, dis_ref, b_ref, out_ref):
    s = aggp_ref[0] + aggp_ref[1]
    out_ref[...] = jnp.maximum(s * dis_ref[...] + b_ref[...], 0.0)


def _finalize(aggp, dis, b2):
    BM = 2000
    grid = (N // BM,)
    return pl.pallas_call(
        _fin_body,
        grid=grid,
        in_specs=[
            pl.BlockSpec((NC, BM, D), lambda i: (0, i, 0)),
            pl.BlockSpec((BM, 1), lambda i: (i, 0)),
            pl.BlockSpec((1, D), lambda i: (0, 0)),
        ],
        out_specs=pl.BlockSpec((BM, D), lambda i: (i, 0)),
        out_shape=jax.ShapeDtypeStruct((N, D), jnp.float32),
    )(aggp, dis, b2)


def kernel(x, edge_index, W, b):
    src = edge_index[0]
    dst = edge_index[1]
    src4 = src.reshape(NW, NCH, CHB, K)
    dst4 = dst.reshape(NW, NCH, CHB, K)
    degp = _deg_kernel(dst)
    degp3 = degp.reshape(NC, N_PAD, 1)[:, :N]
    g, dis = _matmul_scale(x, W, degp3)
    aggp = _agg_kernel(g, src4, dst4)
    return _finalize(aggp, dis, b.reshape(1, D))


# speedup vs baseline: 1.0328x; 1.0328x over previous
import functools

import jax
import jax.numpy as jnp
from jax import lax
from jax.experimental import pallas as pl
from jax.experimental.pallas import tpu as pltpu
from jax.experimental.pallas import tpu_sc as plsc

N = 10000
E = 320000
D = 128

NC = 2
NS = 16
NW = NC * NS
L = 16

K = 125
NB = 80
EPT = NB * K
NCH = 4
CHB = NB // NCH
NT = CHB // 2

N_PAD = 10240
RPT = N_PAD // NS
SR = 64
HR = N_PAD // D

_MESH = plsc.VectorSubcoreMesh(
    core_axis_name="c", subcore_axis_name="s", num_cores=NC, num_subcores=NS)
_SC_PARAMS = pltpu.CompilerParams(needs_layout_passes=False)


@functools.partial(
    pl.kernel,
    out_type=jax.ShapeDtypeStruct((NC, HR, D), jnp.float32),
    mesh=_MESH,
    scratch_types=[
        pltpu.VMEM((HR, D), jnp.float32),
        pltpu.VMEM((EPT,), jnp.int32),
        pltpu.VMEM((HR,), jnp.int32),
        pltpu.VMEM_SHARED((HR, D), jnp.float32),
    ],
    compiler_params=_SC_PARAMS,
)
def _deg_kernel(dst_hbm, out_hbm, hist_v, idx_v, rows_i, hist_sh):
    cid = lax.axis_index("c")
    sid = lax.axis_index("s")
    wid = cid * NS + sid

    zeros16 = jnp.zeros((L,), jnp.float32)
    ones16 = jnp.ones((L,), jnp.float32)
    iota16 = lax.iota(jnp.int32, L)

    def zero_row(r, carry):
        def zero_col(col, c2):
            hist_v[r, pl.ds(col * L, L)] = zeros16
            return c2
        return lax.fori_loop(0, D // L, zero_col, carry)

    lax.fori_loop(0, HR, zero_row, 0)

    def iota_body(i, carry):
        rows_i[pl.ds(i * L, L)] = iota16 + i * L
        return carry

    lax.fori_loop(0, HR // L, iota_body, 0)

    @pl.when(sid == 0)
    def _():
        pltpu.sync_copy(hist_v, hist_sh)

    pltpu.sync_copy(dst_hbm.at[pl.ds(wid * EPT, EPT)], idx_v)

    def acc_body(j, carry):
        idx = idx_v[pl.ds(j * L, L)]
        plsc.addupdate_scatter(hist_v, [idx >> 7, idx & 127], ones16)
        return carry

    lax.fori_loop(0, EPT // L, acc_body, 0)

    plsc.subcore_barrier()
    pltpu.sync_copy(hist_v, hist_sh.at[rows_i], add=True)
    plsc.subcore_barrier()

    @pl.when(sid == 0)
    def _():
        pltpu.sync_copy(hist_sh, out_hbm.at[cid])


def _mm_body(x_ref, w_ref, degp_ref, g_ref, dis_ref):
    deg = jnp.sum(degp_ref[...], axis=0) + 1.0
    dis = lax.rsqrt(deg)
    h = jnp.dot(x_ref[...], w_ref[...], preferred_element_type=jnp.float32)
    g_ref[...] = h * dis
    dis_ref[...] = dis


def _matmul_scale(x, w, degp3):
    BM = 2000
    grid = (N // BM,)
    return pl.pallas_call(
        _mm_body,
        grid=grid,
        in_specs=[
            pl.BlockSpec((BM, D), lambda i: (i, 0)),
            pl.BlockSpec((D, D), lambda i: (0, 0)),
            pl.BlockSpec((NC, BM, 1), lambda i: (0, i, 0)),
        ],
        out_specs=[
            pl.BlockSpec((BM, D), lambda i: (i, 0)),
            pl.BlockSpec((BM, 1), lambda i: (i, 0)),
        ],
        out_shape=[
            jax.ShapeDtypeStruct((N_PAD, D), jnp.float32),
            jax.ShapeDtypeStruct((N, 1), jnp.float32),
        ],
    )(x, w, degp3)


@functools.partial(
    pl.kernel,
    out_type=jax.ShapeDtypeStruct((NC, N_PAD, D), jnp.float32),
    mesh=_MESH,
    scratch_types=[
        pltpu.VMEM_SHARED((N_PAD, D), jnp.float32),
        pltpu.VMEM((CHB, K), jnp.int32),
        pltpu.VMEM((CHB, K), jnp.int32),
        pltpu.VMEM((K, D), jnp.float32),
        pltpu.VMEM((K, D), jnp.float32),
        pltpu.SemaphoreType.DMA,
        pltpu.SemaphoreType.DMA,
    ],
    compiler_params=_SC_PARAMS,
)
def _agg_kernel(g_hbm, src_hbm, dst_hbm, out_hbm,
                acc_sh, src_t, dst_t, rows_a, rows_b, gsem_a, gsem_b):
    cid = lax.axis_index("c")
    sid = lax.axis_index("s")
    wid = cid * NS + sid
    row0 = sid * RPT

    seed_g = jnp.logical_xor(cid == 1, sid < NS // 2)

    @pl.when(seed_g)
    def _():
        for j in range(RPT // SR):
            pltpu.sync_copy(g_hbm.at[pl.ds(row0 + j * SR, SR)],
                            rows_a.at[pl.ds(0, SR)])
            pltpu.sync_copy(rows_a.at[pl.ds(0, SR)],
                            acc_sh.at[pl.ds(row0 + j * SR, SR)])

    @pl.when(jnp.logical_not(seed_g))
    def _():
        zeros16 = jnp.zeros((L,), jnp.float32)

        def zrow(r, carry):
            def zcol(col, c2):
                rows_a[r, pl.ds(col * L, L)] = zeros16
                return c2
            return lax.fori_loop(0, D // L, zcol, carry)

        lax.fori_loop(0, SR, zrow, 0)
        for j in range(RPT // SR):
            pltpu.sync_copy(rows_a.at[pl.ds(0, SR)],
                            acc_sh.at[pl.ds(row0 + j * SR, SR)])

    plsc.subcore_barrier()

    for c in range(NCH):
        pltpu.sync_copy(src_hbm.at[wid, c], src_t)
        pltpu.sync_copy(dst_hbm.at[wid, c], dst_t)
        pltpu.async_copy(g_hbm.at[src_t.at[0]], rows_a, gsem_a)

        def edge_body(t, carry):
            ja = 2 * t
            jb = 2 * t + 1
            pltpu.make_async_copy(
                g_hbm.at[src_t.at[ja]], rows_a, gsem_a).wait()
            pltpu.async_copy(g_hbm.at[src_t.at[jb]], rows_b, gsem_b)
            pltpu.sync_copy(rows_a, acc_sh.at[dst_t.at[ja]], add=True)
            pltpu.make_async_copy(
                g_hbm.at[src_t.at[jb]], rows_b, gsem_b).wait()

            @pl.when(t + 1 < NT)
            def _():
                pltpu.async_copy(g_hbm.at[src_t.at[ja + 2]], rows_a, gsem_a)

            pltpu.sync_copy(rows_b, acc_sh.at[dst_t.at[jb]], add=True)
            return carry

        lax.fori_loop(0, NT, edge_body, 0)

    plsc.subcore_barrier()

    pltpu.sync_copy(acc_sh.at[pl.ds(row0, RPT)],
                    out_hbm.at[cid, pl.ds(row0, RPT)])


def _fin_body(aggp_ref, dis_ref, b_ref, out_ref):
    s = aggp_ref[0] + aggp_ref[1]
    out_ref[...] = jnp.maximum(s * dis_ref[...] + b_ref[...], 0.0)


def _finalize(aggp, dis, b2):
    BM = 2000
    grid = (N // BM,)
    return pl.pallas_call(
        _fin_body,
        grid=grid,
        in_specs=[
            pl.BlockSpec((NC, BM, D), lambda i: (0, i, 0)),
            pl.BlockSpec((BM, 1), lambda i: (i, 0)),
            pl.BlockSpec((1, D), lambda i: (0, 0)),
        ],
        out_specs=pl.BlockSpec((BM, D), lambda i: (i, 0)),
        out_shape=jax.ShapeDtypeStruct((N, D), jnp.float32),
    )(aggp, dis, b2)


def kernel(x, edge_index, W, b):
    src = edge_index[0]
    dst = edge_index[1]
    src4 = src.reshape(NW, NCH, CHB, K)
    dst4 = dst.reshape(NW, NCH, CHB, K)
    degp = _deg_kernel(dst)
    degp3 = degp.reshape(NC, N_PAD, 1)
    g, dis = _matmul_scale(x, W, degp3)
    aggp = _agg_kernel(g, src4, dst4)
    return _finalize(aggp, dis, b.reshape(1, D))


# speedup vs baseline: 1.0357x; 1.0028x over previous
import functools

import jax
import jax.numpy as jnp
from jax import lax
from jax.experimental import pallas as pl
from jax.experimental.pallas import tpu as pltpu
from jax.experimental.pallas import tpu_sc as plsc

N = 10000
E = 320000
D = 128

NC = 2
NS = 16
NW = NC * NS
L = 16

K = 125
NB = 80
EPT = NB * K
NCH = 4
CHB = NB // NCH
NT = CHB // 2

N_PAD = 10240
RPT = N_PAD // NS
SR = 64
HR = N_PAD // D

_MESH = plsc.VectorSubcoreMesh(
    core_axis_name="c", subcore_axis_name="s", num_cores=NC, num_subcores=NS)
_SC_PARAMS = pltpu.CompilerParams(needs_layout_passes=False)


@functools.partial(
    pl.kernel,
    out_type=jax.ShapeDtypeStruct((NC, HR, D), jnp.float32),
    mesh=_MESH,
    scratch_types=[
        pltpu.VMEM((HR, D), jnp.float32),
        pltpu.VMEM((EPT,), jnp.int32),
        pltpu.VMEM((HR,), jnp.int32),
        pltpu.VMEM_SHARED((HR, D), jnp.float32),
    ],
    compiler_params=_SC_PARAMS,
)
def _deg_kernel(dst_hbm, out_hbm, hist_v, idx_v, rows_i, hist_sh):
    cid = lax.axis_index("c")
    sid = lax.axis_index("s")
    wid = cid * NS + sid

    zeros16 = jnp.zeros((L,), jnp.float32)
    ones16 = jnp.ones((L,), jnp.float32)
    iota16 = lax.iota(jnp.int32, L)

    def zero_row(r, carry):
        def zero_col(col, c2):
            hist_v[r, pl.ds(col * L, L)] = zeros16
            return c2
        return lax.fori_loop(0, D // L, zero_col, carry)

    lax.fori_loop(0, HR, zero_row, 0)

    def iota_body(i, carry):
        rows_i[pl.ds(i * L, L)] = iota16 + i * L
        return carry

    lax.fori_loop(0, HR // L, iota_body, 0)

    @pl.when(sid == 0)
    def _():
        pltpu.sync_copy(hist_v, hist_sh)

    pltpu.sync_copy(dst_hbm.at[pl.ds(wid * EPT, EPT)], idx_v)

    def acc_body(j, carry):
        idx = idx_v[pl.ds(j * L, L)]
        plsc.addupdate_scatter(hist_v, [idx >> 7, idx & 127], ones16)
        return carry

    lax.fori_loop(0, EPT // L, acc_body, 0)

    plsc.subcore_barrier()
    pltpu.sync_copy(hist_v, hist_sh.at[rows_i], add=True)
    plsc.subcore_barrier()

    @pl.when(sid == 0)
    def _():
        pltpu.sync_copy(hist_sh, out_hbm.at[cid])


def _mm_body(x_ref, w_ref, degp_ref, g_ref, dis_ref):
    deg = jnp.sum(degp_ref[...], axis=0) + 1.0
    dis = lax.rsqrt(deg)
    h = jnp.dot(x_ref[...], w_ref[...], preferred_element_type=jnp.float32)
    g_ref[...] = h * dis
    dis_ref[...] = dis


def _matmul_scale(x, w, degp3):
    BM = 2000
    grid = (N // BM,)
    return pl.pallas_call(
        _mm_body,
        grid=grid,
        in_specs=[
            pl.BlockSpec((BM, D), lambda i: (i, 0)),
            pl.BlockSpec((D, D), lambda i: (0, 0)),
            pl.BlockSpec((NC, BM, 1), lambda i: (0, i, 0)),
        ],
        out_specs=[
            pl.BlockSpec((BM, D), lambda i: (i, 0)),
            pl.BlockSpec((BM, 1), lambda i: (i, 0)),
        ],
        out_shape=[
            jax.ShapeDtypeStruct((N_PAD, D), jnp.float32),
            jax.ShapeDtypeStruct((N, 1), jnp.float32),
        ],
    )(x, w, degp3)


@functools.partial(
    pl.kernel,
    out_type=jax.ShapeDtypeStruct((NC, N_PAD, D), jnp.float32),
    mesh=_MESH,
    scratch_types=[
        pltpu.VMEM_SHARED((N_PAD, D), jnp.float32),
        pltpu.VMEM((CHB, K), jnp.int32),
        pltpu.VMEM((CHB, K), jnp.int32),
        pltpu.VMEM((K, D), jnp.float32),
        pltpu.VMEM((K, D), jnp.float32),
        pltpu.SemaphoreType.DMA,
        pltpu.SemaphoreType.DMA,
    ],
    compiler_params=_SC_PARAMS,
)
def _agg_kernel(g_hbm, src_hbm, dst_hbm, out_hbm,
                acc_sh, src_t, dst_t, rows_a, rows_b, gsem_a, gsem_b):
    cid = lax.axis_index("c")
    sid = lax.axis_index("s")
    wid = cid * NS + sid
    row0 = sid * RPT

    seed_g = jnp.logical_xor(cid == 1, sid < NS // 2)

    @pl.when(seed_g)
    def _():
        for j in range(RPT // SR):
            pltpu.sync_copy(g_hbm.at[pl.ds(row0 + j * SR, SR)],
                            rows_a.at[pl.ds(0, SR)])
            pltpu.sync_copy(rows_a.at[pl.ds(0, SR)],
                            acc_sh.at[pl.ds(row0 + j * SR, SR)])

    @pl.when(jnp.logical_not(seed_g))
    def _():
        zeros16 = jnp.zeros((L,), jnp.float32)

        def zrow(r, carry):
            def zcol(col, c2):
                rows_a[r, pl.ds(col * L, L)] = zeros16
                return c2
            return lax.fori_loop(0, D // L, zcol, carry)

        lax.fori_loop(0, SR, zrow, 0)
        for j in range(RPT // SR):
            pltpu.sync_copy(rows_a.at[pl.ds(0, SR)],
                            acc_sh.at[pl.ds(row0 + j * SR, SR)])

    plsc.subcore_barrier()

    KH = 64

    def gather2(j, rows, gsem):
        pltpu.async_copy(g_hbm.at[src_t.at[j, pl.ds(0, KH)]],
                         rows.at[pl.ds(0, KH)], gsem)
        pltpu.async_copy(g_hbm.at[src_t.at[j, pl.ds(KH, K - KH)]],
                         rows.at[pl.ds(KH, K - KH)], gsem)

    def wait2(j, rows, gsem):
        pltpu.make_async_copy(g_hbm.at[src_t.at[j, pl.ds(0, KH)]],
                              rows.at[pl.ds(0, KH)], gsem).wait()
        pltpu.make_async_copy(g_hbm.at[src_t.at[j, pl.ds(KH, K - KH)]],
                              rows.at[pl.ds(KH, K - KH)], gsem).wait()

    for c in range(NCH):
        pltpu.sync_copy(src_hbm.at[wid, c], src_t)
        pltpu.sync_copy(dst_hbm.at[wid, c], dst_t)
        gather2(0, rows_a, gsem_a)

        def edge_body(t, carry):
            ja = 2 * t
            jb = 2 * t + 1
            wait2(ja, rows_a, gsem_a)
            gather2(jb, rows_b, gsem_b)
            pltpu.sync_copy(rows_a, acc_sh.at[dst_t.at[ja]], add=True)
            wait2(jb, rows_b, gsem_b)

            @pl.when(t + 1 < NT)
            def _():
                gather2(ja + 2, rows_a, gsem_a)

            pltpu.sync_copy(rows_b, acc_sh.at[dst_t.at[jb]], add=True)
            return carry

        lax.fori_loop(0, NT, edge_body, 0)

    plsc.subcore_barrier()

    pltpu.sync_copy(acc_sh.at[pl.ds(row0, RPT)],
                    out_hbm.at[cid, pl.ds(row0, RPT)])


def _fin_body(aggp_ref, dis_ref, b_ref, out_ref):
    s = aggp_ref[0] + aggp_ref[1]
    out_ref[...] = jnp.maximum(s * dis_ref[...] + b_ref[...], 0.0)


def _finalize(aggp, dis, b2):
    BM = 2000
    grid = (N // BM,)
    return pl.pallas_call(
        _fin_body,
        grid=grid,
        in_specs=[
            pl.BlockSpec((NC, BM, D), lambda i: (0, i, 0)),
            pl.BlockSpec((BM, 1), lambda i: (i, 0)),
            pl.BlockSpec((1, D), lambda i: (0, 0)),
        ],
        out_specs=pl.BlockSpec((BM, D), lambda i: (i, 0)),
        out_shape=jax.ShapeDtypeStruct((N, D), jnp.float32),
    )(aggp, dis, b2)


def kernel(x, edge_index, W, b):
    src = edge_index[0]
    dst = edge_index[1]
    src4 = src.reshape(NW, NCH, CHB, K)
    dst4 = dst.reshape(NW, NCH, CHB, K)
    degp = _deg_kernel(dst)
    degp3 = degp.reshape(NC, N_PAD, 1)
    g, dis = _matmul_scale(x, W, degp3)
    aggp = _agg_kernel(g, src4, dst4)
    return _finalize(aggp, dis, b.reshape(1, D))


# speedup vs baseline: 1.0558x; 1.0195x over previous
import functools

import jax
import jax.numpy as jnp
from jax import lax
from jax.experimental import pallas as pl
from jax.experimental.pallas import tpu as pltpu
from jax.experimental.pallas import tpu_sc as plsc

N = 10000
E = 320000
D = 128

NC = 2
NS = 16
NW = NC * NS
L = 16

K = 125
NB = 80
EPT = NB * K
NCH = 2
CHB = NB // NCH
NT = CHB // 2

N_PAD = 10240
RPT = N_PAD // NS
SR = 64
HR = N_PAD // D

_MESH = plsc.VectorSubcoreMesh(
    core_axis_name="c", subcore_axis_name="s", num_cores=NC, num_subcores=NS)
_SC_PARAMS = pltpu.CompilerParams(needs_layout_passes=False)


@functools.partial(
    pl.kernel,
    out_type=jax.ShapeDtypeStruct((NC, HR, D), jnp.float32),
    mesh=_MESH,
    scratch_types=[
        pltpu.VMEM((HR, D), jnp.float32),
        pltpu.VMEM((EPT,), jnp.int32),
        pltpu.VMEM((HR,), jnp.int32),
        pltpu.VMEM_SHARED((HR, D), jnp.float32),
    ],
    compiler_params=_SC_PARAMS,
)
def _deg_kernel(dst_hbm, out_hbm, hist_v, idx_v, rows_i, hist_sh):
    cid = lax.axis_index("c")
    sid = lax.axis_index("s")
    wid = cid * NS + sid

    zeros16 = jnp.zeros((L,), jnp.float32)
    ones16 = jnp.ones((L,), jnp.float32)
    iota16 = lax.iota(jnp.int32, L)

    def zero_row(r, carry):
        def zero_col(col, c2):
            hist_v[r, pl.ds(col * L, L)] = zeros16
            return c2
        return lax.fori_loop(0, D // L, zero_col, carry)

    lax.fori_loop(0, HR, zero_row, 0)

    def iota_body(i, carry):
        rows_i[pl.ds(i * L, L)] = iota16 + i * L
        return carry

    lax.fori_loop(0, HR // L, iota_body, 0)

    @pl.when(sid == 0)
    def _():
        pltpu.sync_copy(hist_v, hist_sh)

    pltpu.sync_copy(dst_hbm.at[pl.ds(wid * EPT, EPT)], idx_v)

    def acc_body(j, carry):
        idx = idx_v[pl.ds(j * L, L)]
        plsc.addupdate_scatter(hist_v, [idx >> 7, idx & 127], ones16)
        return carry

    lax.fori_loop(0, EPT // L, acc_body, 0)

    plsc.subcore_barrier()
    pltpu.sync_copy(hist_v, hist_sh.at[rows_i], add=True)
    plsc.subcore_barrier()

    @pl.when(sid == 0)
    def _():
        pltpu.sync_copy(hist_sh, out_hbm.at[cid])


def _mm_body(x_ref, w_ref, degp_ref, g_ref, dis_ref):
    deg = jnp.sum(degp_ref[...], axis=0) + 1.0
    dis = lax.rsqrt(deg)
    h = jnp.dot(x_ref[...], w_ref[...], preferred_element_type=jnp.float32)
    g_ref[...] = h * dis
    dis_ref[...] = dis


def _matmul_scale(x, w, degp3):
    BM = 2000
    grid = (N // BM,)
    return pl.pallas_call(
        _mm_body,
        grid=grid,
        in_specs=[
            pl.BlockSpec((BM, D), lambda i: (i, 0)),
            pl.BlockSpec((D, D), lambda i: (0, 0)),
            pl.BlockSpec((NC, BM, 1), lambda i: (0, i, 0)),
        ],
        out_specs=[
            pl.BlockSpec((BM, D), lambda i: (i, 0)),
            pl.BlockSpec((BM, 1), lambda i: (i, 0)),
        ],
        out_shape=[
            jax.ShapeDtypeStruct((N_PAD, D), jnp.float32),
            jax.ShapeDtypeStruct((N, 1), jnp.float32),
        ],
    )(x, w, degp3)


@functools.partial(
    pl.kernel,
    out_type=jax.ShapeDtypeStruct((NC, N_PAD, D), jnp.float32),
    mesh=_MESH,
    scratch_types=[
        pltpu.VMEM_SHARED((N_PAD, D), jnp.float32),
        pltpu.VMEM((CHB, K), jnp.int32),
        pltpu.VMEM((CHB, K), jnp.int32),
        pltpu.VMEM((K, D), jnp.float32),
        pltpu.VMEM((K, D), jnp.float32),
        pltpu.SemaphoreType.DMA,
        pltpu.SemaphoreType.DMA,
    ],
    compiler_params=_SC_PARAMS,
)
def _agg_kernel(g_hbm, src_hbm, dst_hbm, out_hbm,
                acc_sh, src_t, dst_t, rows_a, rows_b, gsem_a, gsem_b):
    cid = lax.axis_index("c")
    sid = lax.axis_index("s")
    wid = cid * NS + sid
    row0 = sid * RPT

    seed_g = jnp.logical_xor(cid == 1, sid < NS // 2)

    @pl.when(seed_g)
    def _():
        for j in range(RPT // SR):
            pltpu.sync_copy(g_hbm.at[pl.ds(row0 + j * SR, SR)],
                            rows_a.at[pl.ds(0, SR)])
            pltpu.sync_copy(rows_a.at[pl.ds(0, SR)],
                            acc_sh.at[pl.ds(row0 + j * SR, SR)])

    @pl.when(jnp.logical_not(seed_g))
    def _():
        zeros16 = jnp.zeros((L,), jnp.float32)

        def zrow(r, carry):
            def zcol(col, c2):
                rows_a[r, pl.ds(col * L, L)] = zeros16
                return c2
            return lax.fori_loop(0, D // L, zcol, carry)

        lax.fori_loop(0, SR, zrow, 0)
        for j in range(RPT // SR):
            pltpu.sync_copy(rows_a.at[pl.ds(0, SR)],
                            acc_sh.at[pl.ds(row0 + j * SR, SR)])

    plsc.subcore_barrier()

    KH = 64

    def gather2(j, rows, gsem):
        pltpu.async_copy(g_hbm.at[src_t.at[j, pl.ds(0, KH)]],
                         rows.at[pl.ds(0, KH)], gsem)
        pltpu.async_copy(g_hbm.at[src_t.at[j, pl.ds(KH, K - KH)]],
                         rows.at[pl.ds(KH, K - KH)], gsem)

    def wait2(j, rows, gsem):
        pltpu.make_async_copy(g_hbm.at[src_t.at[j, pl.ds(0, KH)]],
                              rows.at[pl.ds(0, KH)], gsem).wait()
        pltpu.make_async_copy(g_hbm.at[src_t.at[j, pl.ds(KH, K - KH)]],
                              rows.at[pl.ds(KH, K - KH)], gsem).wait()

    for c in range(NCH):
        pltpu.sync_copy(src_hbm.at[wid, c], src_t)
        pltpu.sync_copy(dst_hbm.at[wid, c], dst_t)
        gather2(0, rows_a, gsem_a)

        def edge_body(t, carry):
            ja = 2 * t
            jb = 2 * t + 1
            wait2(ja, rows_a, gsem_a)
            gather2(jb, rows_b, gsem_b)
            pltpu.sync_copy(rows_a, acc_sh.at[dst_t.at[ja]], add=True)
            wait2(jb, rows_b, gsem_b)

            @pl.when(t + 1 < NT)
            def _():
                gather2(ja + 2, rows_a, gsem_a)

            pltpu.sync_copy(rows_b, acc_sh.at[dst_t.at[jb]], add=True)
            return carry

        lax.fori_loop(0, NT, edge_body, 0)

    plsc.subcore_barrier()

    pltpu.sync_copy(acc_sh.at[pl.ds(row0, RPT)],
                    out_hbm.at[cid, pl.ds(row0, RPT)])


def _fin_body(aggp_ref, dis_ref, b_ref, out_ref):
    s = aggp_ref[0] + aggp_ref[1]
    out_ref[...] = jnp.maximum(s * dis_ref[...] + b_ref[...], 0.0)


def _finalize(aggp, dis, b2):
    BM = 2000
    grid = (N // BM,)
    return pl.pallas_call(
        _fin_body,
        grid=grid,
        in_specs=[
            pl.BlockSpec((NC, BM, D), lambda i: (0, i, 0)),
            pl.BlockSpec((BM, 1), lambda i: (i, 0)),
            pl.BlockSpec((1, D), lambda i: (0, 0)),
        ],
        out_specs=pl.BlockSpec((BM, D), lambda i: (i, 0)),
        out_shape=jax.ShapeDtypeStruct((N, D), jnp.float32),
    )(aggp, dis, b2)


def kernel(x, edge_index, W, b):
    src = edge_index[0]
    dst = edge_index[1]
    src4 = src.reshape(NW, NCH, CHB, K)
    dst4 = dst.reshape(NW, NCH, CHB, K)
    degp = _deg_kernel(dst)
    degp3 = degp.reshape(NC, N_PAD, 1)
    g, dis = _matmul_scale(x, W, degp3)
    aggp = _agg_kernel(g, src4, dst4)
    return _finalize(aggp, dis, b.reshape(1, D))


# speedup vs baseline: 1.0620x; 1.0059x over previous
import functools

import jax
import jax.numpy as jnp
from jax import lax
from jax.experimental import pallas as pl
from jax.experimental.pallas import tpu as pltpu
from jax.experimental.pallas import tpu_sc as plsc

N = 10000
E = 320000
D = 128

NC = 2
NS = 16
NW = NC * NS
L = 16

K = 125
NB = 80
EPT = NB * K
NCH = 2
CHB = NB // NCH
NT = CHB // 2

N_PAD = 10240
RPT = N_PAD // NS
SR = 80
HR = N_PAD // D

_MESH = plsc.VectorSubcoreMesh(
    core_axis_name="c", subcore_axis_name="s", num_cores=NC, num_subcores=NS)
_SC_PARAMS = pltpu.CompilerParams(needs_layout_passes=False)


@functools.partial(
    pl.kernel,
    out_type=jax.ShapeDtypeStruct((NC, HR, D), jnp.float32),
    mesh=_MESH,
    scratch_types=[
        pltpu.VMEM((HR, D), jnp.float32),
        pltpu.VMEM((EPT,), jnp.int32),
        pltpu.VMEM((HR,), jnp.int32),
        pltpu.VMEM_SHARED((HR, D), jnp.float32),
    ],
    compiler_params=_SC_PARAMS,
)
def _deg_kernel(dst_hbm, out_hbm, hist_v, idx_v, rows_i, hist_sh):
    cid = lax.axis_index("c")
    sid = lax.axis_index("s")
    wid = cid * NS + sid

    zeros16 = jnp.zeros((L,), jnp.float32)
    ones16 = jnp.ones((L,), jnp.float32)
    iota16 = lax.iota(jnp.int32, L)

    def zero_row(r, carry):
        def zero_col(col, c2):
            hist_v[r, pl.ds(col * L, L)] = zeros16
            return c2
        return lax.fori_loop(0, D // L, zero_col, carry)

    lax.fori_loop(0, HR, zero_row, 0)

    def iota_body(i, carry):
        rows_i[pl.ds(i * L, L)] = iota16 + i * L
        return carry

    lax.fori_loop(0, HR // L, iota_body, 0)

    @pl.when(sid == 0)
    def _():
        pltpu.sync_copy(hist_v, hist_sh)

    pltpu.sync_copy(dst_hbm.at[pl.ds(wid * EPT, EPT)], idx_v)

    def acc_body(j, carry):
        idx = idx_v[pl.ds(j * L, L)]
        plsc.addupdate_scatter(hist_v, [idx >> 7, idx & 127], ones16)
        return carry

    lax.fori_loop(0, EPT // L, acc_body, 0)

    plsc.subcore_barrier()
    pltpu.sync_copy(hist_v, hist_sh.at[rows_i], add=True)
    plsc.subcore_barrier()

    @pl.when(sid == 0)
    def _():
        pltpu.sync_copy(hist_sh, out_hbm.at[cid])


def _mm_body(x_ref, w_ref, degp_ref, g_ref, dis_ref):
    deg = jnp.sum(degp_ref[...], axis=0) + 1.0
    dis = lax.rsqrt(deg)
    h = jnp.dot(x_ref[...], w_ref[...], preferred_element_type=jnp.float32)
    g_ref[...] = h * dis
    dis_ref[...] = dis


def _matmul_scale(x, w, degp3):
    BM = 2000
    grid = (N // BM,)
    return pl.pallas_call(
        _mm_body,
        grid=grid,
        in_specs=[
            pl.BlockSpec((BM, D), lambda i: (i, 0)),
            pl.BlockSpec((D, D), lambda i: (0, 0)),
            pl.BlockSpec((NC, BM, 1), lambda i: (0, i, 0)),
        ],
        out_specs=[
            pl.BlockSpec((BM, D), lambda i: (i, 0)),
            pl.BlockSpec((BM, 1), lambda i: (i, 0)),
        ],
        out_shape=[
            jax.ShapeDtypeStruct((N_PAD, D), jnp.float32),
            jax.ShapeDtypeStruct((N, 1), jnp.float32),
        ],
    )(x, w, degp3)


@functools.partial(
    pl.kernel,
    out_type=jax.ShapeDtypeStruct((NC, N_PAD, D), jnp.float32),
    mesh=_MESH,
    scratch_types=[
        pltpu.VMEM_SHARED((N_PAD, D), jnp.float32),
        pltpu.VMEM((CHB, K), jnp.int32),
        pltpu.VMEM((CHB, K), jnp.int32),
        pltpu.VMEM((K, D), jnp.float32),
        pltpu.VMEM((K, D), jnp.float32),
        pltpu.SemaphoreType.DMA,
        pltpu.SemaphoreType.DMA,
    ],
    compiler_params=_SC_PARAMS,
)
def _agg_kernel(g_hbm, src_hbm, dst_hbm, out_hbm,
                acc_sh, src_t, dst_t, rows_a, rows_b, gsem_a, gsem_b):
    cid = lax.axis_index("c")
    sid = lax.axis_index("s")
    wid = cid * NS + sid
    row0 = sid * RPT

    seed_g = jnp.logical_xor(cid == 1, sid < NS // 2)

    @pl.when(seed_g)
    def _():
        for j in range(RPT // SR):
            pltpu.sync_copy(g_hbm.at[pl.ds(row0 + j * SR, SR)],
                            rows_a.at[pl.ds(0, SR)])
            pltpu.sync_copy(rows_a.at[pl.ds(0, SR)],
                            acc_sh.at[pl.ds(row0 + j * SR, SR)])

    @pl.when(jnp.logical_not(seed_g))
    def _():
        zeros16 = jnp.zeros((L,), jnp.float32)

        def zrow(r, carry):
            def zcol(col, c2):
                rows_a[r, pl.ds(col * L, L)] = zeros16
                return c2
            return lax.fori_loop(0, D // L, zcol, carry)

        lax.fori_loop(0, SR, zrow, 0)
        for j in range(RPT // SR):
            pltpu.sync_copy(rows_a.at[pl.ds(0, SR)],
                            acc_sh.at[pl.ds(row0 + j * SR, SR)])

    plsc.subcore_barrier()

    KH = 64

    def gather2(j, rows, gsem):
        pltpu.async_copy(g_hbm.at[src_t.at[j, pl.ds(0, KH)]],
                         rows.at[pl.ds(0, KH)], gsem)
        pltpu.async_copy(g_hbm.at[src_t.at[j, pl.ds(KH, K - KH)]],
                         rows.at[pl.ds(KH, K - KH)], gsem)

    def wait2(j, rows, gsem):
        pltpu.make_async_copy(g_hbm.at[src_t.at[j, pl.ds(0, KH)]],
                              rows.at[pl.ds(0, KH)], gsem).wait()
        pltpu.make_async_copy(g_hbm.at[src_t.at[j, pl.ds(KH, K - KH)]],
                              rows.at[pl.ds(KH, K - KH)], gsem).wait()

    for c in range(NCH):
        pltpu.sync_copy(src_hbm.at[wid, c], src_t)
        pltpu.sync_copy(dst_hbm.at[wid, c], dst_t)
        gather2(0, rows_a, gsem_a)

        def edge_body(t, carry):
            ja = 2 * t
            jb = 2 * t + 1
            wait2(ja, rows_a, gsem_a)
            gather2(jb, rows_b, gsem_b)
            pltpu.sync_copy(rows_a, acc_sh.at[dst_t.at[ja]], add=True)
            wait2(jb, rows_b, gsem_b)

            @pl.when(t + 1 < NT)
            def _():
                gather2(ja + 2, rows_a, gsem_a)

            pltpu.sync_copy(rows_b, acc_sh.at[dst_t.at[jb]], add=True)
            return carry

        lax.fori_loop(0, NT, edge_body, 0)

    plsc.subcore_barrier()

    pltpu.sync_copy(acc_sh.at[pl.ds(row0, RPT)],
                    out_hbm.at[cid, pl.ds(row0, RPT)])


def _fin_body(aggp_ref, dis_ref, b_ref, out_ref):
    s = aggp_ref[0] + aggp_ref[1]
    out_ref[...] = jnp.maximum(s * dis_ref[...] + b_ref[...], 0.0)


def _finalize(aggp, dis, b2):
    BM = 2000
    grid = (N // BM,)
    return pl.pallas_call(
        _fin_body,
        grid=grid,
        in_specs=[
            pl.BlockSpec((NC, BM, D), lambda i: (0, i, 0)),
            pl.BlockSpec((BM, 1), lambda i: (i, 0)),
            pl.BlockSpec((1, D), lambda i: (0, 0)),
        ],
        out_specs=pl.BlockSpec((BM, D), lambda i: (i, 0)),
        out_shape=jax.ShapeDtypeStruct((N, D), jnp.float32),
    )(aggp, dis, b2)


def kernel(x, edge_index, W, b):
    src = edge_index[0]
    dst = edge_index[1]
    src4 = src.reshape(NW, NCH, CHB, K)
    dst4 = dst.reshape(NW, NCH, CHB, K)
    degp = _deg_kernel(dst)
    degp3 = degp.reshape(NC, N_PAD, 1)
    g, dis = _matmul_scale(x, W, degp3)
    aggp = _agg_kernel(g, src4, dst4)
    return _finalize(aggp, dis, b.reshape(1, D))
